# baseline (speedup 1.0000x reference)
import functools

import jax
import jax.numpy as jnp
from jax import lax
from jax.experimental import pallas as pl
from jax.experimental.pallas import tpu as pltpu
from jax.experimental.pallas import tpu_sc as plsc

_NC = 1
_NS = 16
_NW = _NC * _NS
_GROUP = 16
_LANES = 16

_TV = 16384
_NBUF = 4


def _sc_gather_sum(idx, table):
    n = idx.shape[0]
    vocab, d = table.shape
    mesh = plsc.VectorSubcoreMesh(
        core_axis_name="c", subcore_axis_name="s", num_cores=_NC)

    def body(idx_hbm, table_hbm, out_hbm, idx_v, rows_v, acc_v, sem):
        wid = lax.axis_index("s") * _NC + lax.axis_index("c")
        base = wid * _GROUP
        off1 = jnp.minimum(base, n - 8)
        off2 = jnp.minimum(base + 8, n - 8)
        pltpu.sync_copy(idx_hbm.at[pl.ds(off1, 8)], idx_v.at[pl.ds(0, 8)])
        pltpu.sync_copy(idx_hbm.at[pl.ds(off2, 8)], idx_v.at[pl.ds(8, 8)])
        pltpu.async_copy(table_hbm.at[idx_v], rows_v, sem).wait()
        for c in range(d // _LANES):
            sl = pl.ds(c * _LANES, _LANES)
            acc = jnp.zeros((_LANES,), jnp.float32)
            for r in range(_GROUP):
                w_r = jnp.where(base + r < n, 1.0, 0.0)
                acc = acc + rows_v[r, sl] * w_r
            acc_v[sl] = acc
        pltpu.sync_copy(acc_v, out_hbm.at[wid])

    run = pl.kernel(
        body,
        out_type=jax.ShapeDtypeStruct((_NW, d), jnp.float32),
        mesh=mesh,
        scratch_types=[
            pltpu.VMEM((_GROUP,), jnp.int32),
            pltpu.VMEM((_GROUP, d), jnp.float32),
            pltpu.VMEM((d,), jnp.float32),
            pltpu.SemaphoreType.DMA,
        ],
    )
    return run(idx, table)


def _scalar_exp(x):
    return jnp.max(jnp.exp(jnp.full((1, 128), x)))


def _scalar_log(x):
    return jnp.max(jnp.log(jnp.full((1, 128), x)))


def _a_body(nt, vocab, p_ref, wp_ref, bp_ref, wo_hbm, bo_ref, out_ref, *scr):
    bufs = scr[:_NBUF]
    sems = scr[_NBUF:]
    tv = _TV
    half = tv // 2

    def tile_copies(t):
        sz = min(vocab - t * tv, tv)
        out = []
        for j, (o, w) in enumerate(((0, min(sz, half)),
                                    (half, max(0, sz - half)))):
            if w > 0:
                out.append(pltpu.make_async_copy(
                    wo_hbm.at[pl.ds(t * tv + o, w)],
                    bufs[t % _NBUF].at[pl.ds(o, w)],
                    sems[2 * (t % _NBUF) + j],
                ))
        return out

    def start_tile(t):
        for c in tile_copies(t):
            c.start()

    def wait_tile(t):
        for c in tile_copies(t):
            c.wait()

    for t in range(min(_NBUF - 1, nt)):
        start_tile(t)

    e = jnp.sum(p_ref[...], axis=0, keepdims=True)
    h = jnp.dot(e, wp_ref[...], preferred_element_type=jnp.float32)
    h = jnp.maximum(h + bp_ref[...], 0.0)
    hT = jnp.transpose(h)
    h8 = jnp.broadcast_to(hT, (hT.shape[0], 8))

    m = jnp.float32(-jnp.inf)
    s = jnp.float32(0.0)
    for t in range(nt):
        if t + _NBUF - 1 < nt:
            start_tile(t + _NBUF - 1)
        wait_tile(t)
        sz = min(vocab - t * tv, tv)
        wo = bufs[t % _NBUF][pl.ds(0, sz), :]
        lt = jnp.dot(wo, h8, preferred_element_type=jnp.float32)
        lr = jnp.transpose(lt)[0:1, :] + bo_ref[:, t * tv:t * tv + sz]
        out_ref[:, t * tv:t * tv + sz] = lr
        m_new = jnp.maximum(m, jnp.max(lr))
        s = s * _scalar_exp(m - m_new) + jnp.sum(jnp.exp(lr - m_new))
        m = m_new

    lse = m + _scalar_log(s)
    out_ref[...] = out_ref[...] - lse


def _tc_mlp_logsoftmax(partials, W_proj, bp2, woT, bo2):
    vocab, d = woT.shape
    nt = -(-vocab // _TV)

    return pl.pallas_call(
        functools.partial(_a_body, nt, vocab),
        in_specs=[
            pl.BlockSpec((_NW, d), lambda: (0, 0)),
            pl.BlockSpec((d, d), lambda: (0, 0)),
            pl.BlockSpec((1, d), lambda: (0, 0)),
            pl.BlockSpec(memory_space=pl.ANY),
            pl.BlockSpec((1, vocab), lambda: (0, 0)),
        ],
        out_specs=pl.BlockSpec((1, vocab), lambda: (0, 0)),
        out_shape=jax.ShapeDtypeStruct((1, vocab), jnp.float32),
        scratch_shapes=(
            [pltpu.VMEM((_TV, d), jnp.float32) for _ in range(_NBUF)]
            + [pltpu.SemaphoreType.DMA for _ in range(2 * _NBUF)]
        ),
    )(partials, W_proj, bp2, woT, bo2)


def kernel(inputs, table, W_proj, b_proj, W_out, b_out):
    idx = inputs.astype(jnp.int32)
    partials = _sc_gather_sum(idx, table)
    return _tc_mlp_logsoftmax(
        partials,
        W_proj,
        b_proj.reshape(1, -1),
        W_out.T,
        b_out.reshape(1, -1),
    )


# speedup vs baseline: 1.0850x; 1.0850x over previous
import functools

import jax
import jax.numpy as jnp
from jax import lax
from jax.experimental import pallas as pl
from jax.experimental.pallas import tpu as pltpu
from jax.experimental.pallas import tpu_sc as plsc

_NC = 1
_NS = 16
_NW = _NC * _NS
_GROUP = 16
_LANES = 16

_TV = 16384
_NBUF = 4


def _sc_gather(idx, table):
    n = idx.shape[0]
    vocab, d = table.shape
    mesh = plsc.VectorSubcoreMesh(
        core_axis_name="c", subcore_axis_name="s", num_cores=_NC)

    def body(idx_hbm, table_hbm, out_hbm, idx_v, rows_v, sem):
        wid = lax.axis_index("s") * _NC + lax.axis_index("c")
        base = wid * _GROUP
        off1 = jnp.minimum(base, n - 8)
        off2 = jnp.minimum(base + 8, n - 8)
        pltpu.sync_copy(idx_hbm.at[pl.ds(off1, 8)], idx_v.at[pl.ds(0, 8)])
        pltpu.sync_copy(idx_hbm.at[pl.ds(off2, 8)], idx_v.at[pl.ds(8, 8)])
        pltpu.async_copy(table_hbm.at[idx_v], rows_v, sem).wait()
        pltpu.sync_copy(rows_v, out_hbm.at[pl.ds(base, _GROUP)])

    run = pl.kernel(
        body,
        out_type=jax.ShapeDtypeStruct((_NW * _GROUP, d), jnp.float32),
        mesh=mesh,
        scratch_types=[
            pltpu.VMEM((_GROUP,), jnp.int32),
            pltpu.VMEM((_GROUP, d), jnp.float32),
            pltpu.SemaphoreType.DMA,
        ],
    )
    return run(idx, table)


def _scalar_exp(x):
    return jnp.max(jnp.exp(jnp.full((1, 128), x)))


def _scalar_log(x):
    return jnp.max(jnp.log(jnp.full((1, 128), x)))


def _a_body(nt, vocab, n, p_ref, wp_ref, bp_ref, wo_hbm, bo_ref, out_ref,
            *scr):
    bufs = scr[:_NBUF]
    sems = scr[_NBUF:]
    tv = _TV

    def tile_copy(t):
        sz = min(vocab - t * tv, tv)
        return pltpu.make_async_copy(
            wo_hbm.at[pl.ds(t * tv, sz)],
            bufs[t % _NBUF].at[pl.ds(0, sz)],
            sems[t % _NBUF],
        )

    def start_tile(t):
        tile_copy(t).start()

    def wait_tile(t):
        tile_copy(t).wait()

    for t in range(min(_NBUF - 1, nt)):
        start_tile(t)

    rown = lax.broadcasted_iota(jnp.int32, (p_ref.shape[0], 1), 0)
    rows = jnp.where(rown < n, p_ref[...], 0.0)
    e = jnp.sum(rows, axis=0, keepdims=True)
    h = jnp.dot(e, wp_ref[...], preferred_element_type=jnp.float32)
    h = jnp.maximum(h + bp_ref[...], 0.0)
    hT = jnp.transpose(h)
    h8 = jnp.broadcast_to(hT, (hT.shape[0], 8))

    m = jnp.float32(-jnp.inf)
    s = jnp.float32(0.0)
    for t in range(nt):
        if t + _NBUF - 1 < nt:
            start_tile(t + _NBUF - 1)
        wait_tile(t)
        sz = min(vocab - t * tv, tv)
        wo = bufs[t % _NBUF][pl.ds(0, sz), :]
        lt = jnp.dot(wo, h8, preferred_element_type=jnp.float32)
        lr = jnp.transpose(lt)[0:1, :] + bo_ref[:, t * tv:t * tv + sz]
        out_ref[:, t * tv:t * tv + sz] = lr
        m_new = jnp.maximum(m, jnp.max(lr))
        s = s * _scalar_exp(m - m_new) + jnp.sum(jnp.exp(lr - m_new))
        m = m_new

    lse = m + _scalar_log(s)
    out_ref[...] = out_ref[...] - lse


def _tc_mlp_logsoftmax(rows, n, W_proj, bp2, woT, bo2):
    vocab, d = woT.shape
    nt = -(-vocab // _TV)

    return pl.pallas_call(
        functools.partial(_a_body, nt, vocab, n),
        in_specs=[
            pl.BlockSpec((_NW * _GROUP, d), lambda: (0, 0)),
            pl.BlockSpec((d, d), lambda: (0, 0)),
            pl.BlockSpec((1, d), lambda: (0, 0)),
            pl.BlockSpec(memory_space=pl.ANY),
            pl.BlockSpec((1, vocab), lambda: (0, 0)),
        ],
        out_specs=pl.BlockSpec((1, vocab), lambda: (0, 0)),
        out_shape=jax.ShapeDtypeStruct((1, vocab), jnp.float32),
        scratch_shapes=(
            [pltpu.VMEM((_TV, d), jnp.float32) for _ in range(_NBUF)]
            + [pltpu.SemaphoreType.DMA for _ in range(_NBUF)]
        ),
    )(rows, W_proj, bp2, woT, bo2)


def kernel(inputs, table, W_proj, b_proj, W_out, b_out):
    idx = inputs.astype(jnp.int32)
    rows = _sc_gather(idx, table)
    return _tc_mlp_logsoftmax(
        rows,
        idx.shape[0],
        W_proj,
        b_proj.reshape(1, -1),
        W_out.T,
        b_out.reshape(1, -1),
    )


# speedup vs baseline: 1.0884x; 1.0031x over previous
import functools

import jax
import jax.numpy as jnp
from jax import lax
from jax.experimental import pallas as pl
from jax.experimental.pallas import tpu as pltpu
from jax.experimental.pallas import tpu_sc as plsc

_NC = 1
_NS = 16
_NW = _NC * _NS
_GROUP = 16
_LANES = 16

_TV = 16384
_NBUF = 4


def _tile_plan(vocab):
    sizes = []
    for sz in (2048, 4096, 8192):
        if sum(sizes) + sz <= vocab:
            sizes.append(sz)
    while vocab - sum(sizes) >= _TV:
        sizes.append(_TV)
    if vocab > sum(sizes):
        sizes.append(vocab - sum(sizes))
    offs = [sum(sizes[:t]) for t in range(len(sizes))]
    return sizes, offs


def _sc_gather(idx, table):
    n = idx.shape[0]
    vocab, d = table.shape
    mesh = plsc.VectorSubcoreMesh(
        core_axis_name="c", subcore_axis_name="s", num_cores=_NC)

    def body(idx_hbm, table_hbm, out_hbm, idx_v, rows_v, sem):
        wid = lax.axis_index("s") * _NC + lax.axis_index("c")
        base = wid * _GROUP
        off1 = jnp.minimum(base, n - 8)
        off2 = jnp.minimum(base + 8, n - 8)
        pltpu.sync_copy(idx_hbm.at[pl.ds(off1, 8)], idx_v.at[pl.ds(0, 8)])
        pltpu.sync_copy(idx_hbm.at[pl.ds(off2, 8)], idx_v.at[pl.ds(8, 8)])
        pltpu.async_copy(table_hbm.at[idx_v], rows_v, sem).wait()
        pltpu.sync_copy(rows_v, out_hbm.at[pl.ds(base, _GROUP)])

    run = pl.kernel(
        body,
        out_type=jax.ShapeDtypeStruct((_NW * _GROUP, d), jnp.float32),
        mesh=mesh,
        scratch_types=[
            pltpu.VMEM((_GROUP,), jnp.int32),
            pltpu.VMEM((_GROUP, d), jnp.float32),
            pltpu.SemaphoreType.DMA,
        ],
    )
    return run(idx, table)


def _scalar_exp(x):
    return jnp.max(jnp.exp(jnp.full((1, 128), x)))


def _scalar_log(x):
    return jnp.max(jnp.log(jnp.full((1, 128), x)))


def _a_body(sizes, offs, vocab, n, p_ref, wp_ref, bp_ref, wo_hbm, bo_ref,
            out_ref, *scr):
    bufs = scr[:_NBUF]
    sems = scr[_NBUF:]
    nt = len(sizes)

    def tile_copy(t):
        return pltpu.make_async_copy(
            wo_hbm.at[pl.ds(offs[t], sizes[t])],
            bufs[t % _NBUF].at[pl.ds(0, sizes[t])],
            sems[t % _NBUF],
        )

    def start_tile(t):
        tile_copy(t).start()

    def wait_tile(t):
        tile_copy(t).wait()

    for t in range(min(_NBUF - 1, nt)):
        start_tile(t)

    rown = lax.broadcasted_iota(jnp.int32, (p_ref.shape[0], 1), 0)
    rows = jnp.where(rown < n, p_ref[...], 0.0)
    e = jnp.sum(rows, axis=0, keepdims=True)
    h = jnp.dot(e, wp_ref[...], preferred_element_type=jnp.float32)
    h = jnp.maximum(h + bp_ref[...], 0.0)
    hT = jnp.transpose(h)
    h8 = jnp.broadcast_to(hT, (hT.shape[0], 8))

    m = jnp.float32(-jnp.inf)
    s = jnp.float32(0.0)
    for t in range(nt):
        if t + _NBUF - 1 < nt:
            start_tile(t + _NBUF - 1)
        wait_tile(t)
        sz, off = sizes[t], offs[t]
        wo = bufs[t % _NBUF][pl.ds(0, sz), :]
        lt = jnp.dot(wo, h8, preferred_element_type=jnp.float32)
        lr = jnp.transpose(lt)[0:1, :] + bo_ref[:, off:off + sz]
        out_ref[:, off:off + sz] = lr
        m_new = jnp.maximum(m, jnp.max(lr))
        s = s * _scalar_exp(m - m_new) + jnp.sum(jnp.exp(lr - m_new))
        m = m_new

    lse = m + _scalar_log(s)
    out_ref[...] = out_ref[...] - lse


def _tc_mlp_logsoftmax(rows, n, W_proj, bp2, woT, bo2):
    vocab, d = woT.shape
    sizes, offs = _tile_plan(vocab)

    return pl.pallas_call(
        functools.partial(_a_body, sizes, offs, vocab, n),
        in_specs=[
            pl.BlockSpec((_NW * _GROUP, d), lambda: (0, 0)),
            pl.BlockSpec((d, d), lambda: (0, 0)),
            pl.BlockSpec((1, d), lambda: (0, 0)),
            pl.BlockSpec(memory_space=pl.ANY),
            pl.BlockSpec((1, vocab), lambda: (0, 0)),
        ],
        out_specs=pl.BlockSpec((1, vocab), lambda: (0, 0)),
        out_shape=jax.ShapeDtypeStruct((1, vocab), jnp.float32),
        scratch_shapes=(
            [pltpu.VMEM((_TV, d), jnp.float32) for _ in range(_NBUF)]
            + [pltpu.SemaphoreType.DMA for _ in range(_NBUF)]
        ),
    )(rows, W_proj, bp2, woT, bo2)


def kernel(inputs, table, W_proj, b_proj, W_out, b_out):
    idx = inputs.astype(jnp.int32)
    rows = _sc_gather(idx, table)
    return _tc_mlp_logsoftmax(
        rows,
        idx.shape[0],
        W_proj,
        b_proj.reshape(1, -1),
        W_out.T,
        b_out.reshape(1, -1),
    )
